# baseline (speedup 1.0000x reference)
import functools

import jax
import jax.numpy as jnp
import numpy as np
from jax import lax
from jax.experimental import pallas as pl
from jax.experimental.pallas import tpu as pltpu
from jax.experimental.pallas import tpu_sc as plsc

DIM = 1024
N_TOKENS = 10
BATCH = 4
SEQ = 2058
SEQ_PAD = 2064
N_GATHER = BATCH * (SEQ - N_TOKENS)

NC = 2
NS = 16
NW = NC * NS
ROWS_PER_W = N_GATHER // NW
CHUNK = 32
N_CHUNKS = ROWS_PER_W // CHUNK
W_PER_BATCH = NW // BATCH

_SIZES = (8, 24) + (CHUNK,) * ((ROWS_PER_W - 32) // CHUNK)
_OFFS = tuple(np.cumsum((0,) + _SIZES[:-1]).tolist())
N_CHUNKS_T = len(_SIZES)

_OBASE = (N_TOKENS
          + (np.arange(NW, dtype=np.int32) % W_PER_BATCH)[:, None, None]
          * ROWS_PER_W)
_OIDX8 = _OBASE + np.arange(8, dtype=np.int32)[None, None, :]
_OIDX24 = _OBASE + 8 + np.arange(24, dtype=np.int32)[None, None, :]
_OIDX = (_OBASE + 32
         + (np.arange(N_CHUNKS_T - 2, dtype=np.int32) * CHUNK)[:, None]
         + np.arange(CHUNK, dtype=np.int32)[None, None, :])


@functools.partial(
    pl.kernel,
    mesh=plsc.VectorSubcoreMesh(
        core_axis_name="c", subcore_axis_name="s", num_cores=NC),
    out_type=jax.ShapeDtypeStruct((BATCH, SEQ_PAD, DIM), jnp.float32),
    scratch_types=[
        pltpu.VMEM((ROWS_PER_W,), jnp.int32),
        pltpu.VMEM((N_CHUNKS - 1, CHUNK), jnp.int32),
        pltpu.VMEM((1, 8), jnp.int32),
        pltpu.VMEM((1, 24), jnp.int32),
        pltpu.VMEM((CHUNK, DIM), jnp.float32),
        pltpu.VMEM((CHUNK, DIM), jnp.float32),
        pltpu.VMEM((CHUNK, DIM), jnp.float32),
        pltpu.VMEM((16, DIM), jnp.float32),
    ] + [pltpu.SemaphoreType.DMA] * 7,
)
def _soft_embed(table_hbm, idx_hbm, learned_hbm, oidx_hbm, oidx8_hbm,
                oidx24_hbm, out_hbm,
                idx_v, oidx_v, oidx8_v, oidx24_v, buf0, buf1, buf2, lbuf,
                gsem0, gsem1, gsem2, ssem0, ssem1, ssem2, lsem):
    wid = lax.axis_index("s") * NC + lax.axis_index("c")
    gbase = wid * ROWS_PER_W
    b = wid // W_PER_BATCH
    is_lworker = wid % W_PER_BATCH == 0

    lane = lax.iota(jnp.int32, 16)
    lsrc = jnp.minimum(lane, N_TOKENS - 1)
    ltgt = jnp.where(lane < N_TOKENS, lane, lane + (SEQ - N_TOKENS))

    @pl.when(is_lworker)
    def _():
        pltpu.async_copy(learned_hbm.at[lsrc], lbuf, lsem)

    pltpu.sync_copy(idx_hbm.at[pl.ds(gbase, ROWS_PER_W)], idx_v)

    NBUF = 3
    bufs = (buf0, buf1, buf2)
    gsems = (gsem0, gsem1, gsem2)
    ssems = (ssem0, ssem1, ssem2)

    def buf_slice(j):
        buf = bufs[j % NBUF]
        n = _SIZES[j]
        return buf if n == CHUNK else buf.at[pl.ds(0, n)]

    def out_idx(j):
        if j == 0:
            return oidx8_v.at[0]
        if j == 1:
            return oidx24_v.at[0]
        return oidx_v.at[j - 2]

    def start_gather(j):
        return pltpu.async_copy(
            table_hbm.at[idx_v.at[pl.ds(_OFFS[j], _SIZES[j])]],
            buf_slice(j), gsems[j % NBUF])

    def start_scatter(j):
        return pltpu.async_copy(
            buf_slice(j), out_hbm.at[b].at[out_idx(j)], ssems[j % NBUF])

    gcopies = [None] * N_CHUNKS_T
    scopies = [None] * N_CHUNKS_T
    for j in range(min(NBUF, N_CHUNKS_T)):
        gcopies[j] = start_gather(j)

    pltpu.sync_copy(oidx8_hbm.at[wid], oidx8_v)
    pltpu.sync_copy(oidx24_hbm.at[wid], oidx24_v)
    pltpu.sync_copy(oidx_hbm.at[wid], oidx_v)

    @pl.when(is_lworker)
    def _():
        pltpu.make_async_copy(learned_hbm.at[lsrc], lbuf, lsem).wait()
        pltpu.async_copy(lbuf, out_hbm.at[b].at[ltgt], lsem)

    for j in range(N_CHUNKS_T):
        gcopies[j].wait()
        scopies[j] = start_scatter(j)
        nxt = j + NBUF - 1
        if j >= 1 and nxt < N_CHUNKS_T:
            scopies[j - 1].wait()
            gcopies[nxt] = start_gather(nxt)
    for j in range(max(0, N_CHUNKS_T - NBUF), N_CHUNKS_T):
        scopies[j].wait()

    @pl.when(is_lworker)
    def _():
        pltpu.make_async_copy(lbuf, out_hbm.at[b].at[ltgt], lsem).wait()


def kernel(tokens, wte_weight, learned_embedding):
    idx = tokens[:, N_TOKENS:].reshape(-1).astype(jnp.int32)
    out = _soft_embed(wte_weight, idx, learned_embedding, _OIDX, _OIDX8,
                      _OIDX24)
    return out[:, :SEQ, :]


# speedup vs baseline: 1.0354x; 1.0354x over previous
import functools

import jax
import jax.numpy as jnp
import numpy as np
from jax import lax
from jax.experimental import pallas as pl
from jax.experimental.pallas import tpu as pltpu
from jax.experimental.pallas import tpu_sc as plsc

DIM = 1024
N_TOKENS = 10
BATCH = 4
SEQ = 2058
SEQ_PAD = 2064
N_GATHER = BATCH * (SEQ - N_TOKENS)

NC = 2
NS = 16
NW = NC * NS
ROWS_PER_W = N_GATHER // NW
CHUNK = 32
N_CHUNKS = ROWS_PER_W // CHUNK
W_PER_BATCH = NW // BATCH

_OIDX = (N_TOKENS + (np.arange(NW, dtype=np.int32) % W_PER_BATCH)[:, None, None]
         * ROWS_PER_W
         + (np.arange(N_CHUNKS, dtype=np.int32) * CHUNK)[None, :, None]
         + np.arange(CHUNK, dtype=np.int32)[None, None, :])


@functools.partial(
    pl.kernel,
    mesh=plsc.VectorSubcoreMesh(
        core_axis_name="c", subcore_axis_name="s", num_cores=NC),
    out_type=jax.ShapeDtypeStruct((BATCH, SEQ_PAD, DIM), jnp.float32),
    scratch_types=[
        pltpu.VMEM((ROWS_PER_W,), jnp.int32),
        pltpu.VMEM((N_CHUNKS, CHUNK), jnp.int32),
        pltpu.VMEM((CHUNK, DIM), jnp.float32),
        pltpu.VMEM((CHUNK, DIM), jnp.float32),
        pltpu.VMEM((CHUNK, DIM), jnp.float32),
        pltpu.VMEM((16, DIM), jnp.float32),
    ] + [pltpu.SemaphoreType.DMA] * 7,
)
def _soft_embed(table_hbm, idx_hbm, learned_hbm, oidx_hbm, out_hbm,
                idx_v, oidx_v, buf0, buf1, buf2, lbuf,
                gsem0, gsem1, gsem2, ssem0, ssem1, ssem2, lsem):
    wid = lax.axis_index("s") * NC + lax.axis_index("c")
    gbase = wid * ROWS_PER_W
    b = wid // W_PER_BATCH
    is_lworker = wid % W_PER_BATCH == 0

    lane = lax.iota(jnp.int32, 16)
    lsrc = jnp.minimum(lane, N_TOKENS - 1)
    ltgt = jnp.where(lane < N_TOKENS, lane, lane + (SEQ - N_TOKENS))

    @pl.when(is_lworker)
    def _():
        pltpu.async_copy(learned_hbm.at[lsrc], lbuf, lsem)

    pltpu.sync_copy(idx_hbm.at[pl.ds(gbase, ROWS_PER_W)], idx_v)

    NBUF = 3
    bufs = (buf0, buf1, buf2)
    gsems = (gsem0, gsem1, gsem2)
    ssems = (ssem0, ssem1, ssem2)

    def start_gather(j):
        return pltpu.async_copy(
            table_hbm.at[idx_v.at[pl.ds(j * CHUNK, CHUNK)]],
            bufs[j % NBUF], gsems[j % NBUF])

    def start_scatter(j):
        return pltpu.async_copy(
            bufs[j % NBUF], out_hbm.at[b].at[oidx_v.at[j]], ssems[j % NBUF])

    gcopies = [None] * N_CHUNKS
    scopies = [None] * N_CHUNKS
    for j in range(min(NBUF, N_CHUNKS)):
        gcopies[j] = start_gather(j)

    pltpu.sync_copy(oidx_hbm.at[wid], oidx_v)

    @pl.when(is_lworker)
    def _():
        pltpu.make_async_copy(learned_hbm.at[lsrc], lbuf, lsem).wait()
        pltpu.async_copy(lbuf, out_hbm.at[b].at[ltgt], lsem)

    for j in range(N_CHUNKS):
        gcopies[j].wait()
        scopies[j] = start_scatter(j)
        nxt = j + NBUF - 1
        if j >= 1 and nxt < N_CHUNKS:
            scopies[j - 1].wait()
            gcopies[nxt] = start_gather(nxt)
    for j in range(max(0, N_CHUNKS - NBUF), N_CHUNKS):
        scopies[j].wait()

    @pl.when(is_lworker)
    def _():
        pltpu.make_async_copy(lbuf, out_hbm.at[b].at[ltgt], lsem).wait()


def kernel(tokens, wte_weight, learned_embedding):
    idx = tokens[:, N_TOKENS:].reshape(-1).astype(jnp.int32)
    out = _soft_embed(wte_weight, idx, learned_embedding, _OIDX)
    return out[:, :SEQ, :]
